# baseline (speedup 1.0000x reference)
import functools

import jax
import jax.numpy as jnp
from jax import lax
from jax.experimental import pallas as pl
from jax.experimental.pallas import tpu as pltpu
from jax.experimental.pallas import tpu_sc as plsc

NC = 2
NS = 16
L = 16
NW = NC * NS
DIM = 64
CTX = 200
TOK = CTX + 1
HID = 128
IN_DIM = DIM * TOK + 1
TPW = 7
ROWS = TPW * DIM
PAD = 240


def _sc_body(category_h, title_h, emb_h, emb2_h, w1_h, out_h,
             title_v, cat_v, rows_v, w1_v, acc_v, sem_a, sem_b):
    c = lax.axis_index("c")
    s = lax.axis_index("s")
    wid = s * NC + c
    tb = jnp.minimum(TPW * wid, TOK - TPW)

    for q in range(PAD // L):
        title_v[pl.ds(q * L, L)] = jnp.zeros((L,), jnp.int32)
    pltpu.sync_copy(title_h, title_v.at[pl.ds(L, CTX)])
    tv = title_v[pl.ds(tb + (L - 1), L)]

    rv = tv >> 1
    offv = (tv & 1) << 6

    pltpu.sync_copy(category_h, cat_v.at[pl.ds(0, 1)])
    cv = cat_v[pl.ds(0, L)]
    is_w0 = wid == 0
    off0 = jnp.where(is_w0, (cv[0] & 1) << 6, offv[0])

    copies = [pltpu.async_copy(emb_h.at[pl.ds(rv[k], 1)],
                               rows_v.at[pl.ds(k, 1)], sem_a)
              for k in range(TPW)]
    for cp in copies:
        cp.wait()

    @pl.when(is_w0)
    def _():
        pltpu.async_copy(emb2_h.at[pl.ds(cv[0] >> 1, 1)],
                         rows_v.at[pl.ds(TPW, 1)], sem_b).wait()
        for q in range(HID // L):
            rows_v[0, pl.ds(q * L, L)] = rows_v[TPW, pl.ds(q * L, L)]

    pltpu.sync_copy(w1_h.at[pl.ds(DIM * tb, ROWS)], w1_v)

    hi = jnp.minimum(TPW * wid + TPW, TOK)
    for k in range(TPW):
        t = tb + k
        scale = jnp.where((t >= TPW * wid) & (t < hi), 1.0, 0.0).astype(jnp.float32)
        for q in range(HID // L):
            rows_v[k, pl.ds(q * L, L)] = rows_v[k, pl.ds(q * L, L)] * scale

    acc = tuple(jnp.zeros((L,), jnp.float32) for _ in range(HID // L))
    for k in range(TPW):
        off_k = off0 if k == 0 else offv[k]

        def body(db, a, k=k, off_k=off_k):
            xv = rows_v[k, pl.ds(off_k + db * L, L)]
            for e in range(L):
                xb = jnp.broadcast_to(xv[e], (L,))
                i = k * DIM + db * L + e
                a = tuple(a[j] + xb * w1_v[i, pl.ds(j * L, L)]
                          for j in range(HID // L))
            return a

        acc = lax.fori_loop(0, DIM // L, body, acc)
    for j in range(HID // L):
        acc_v[0, pl.ds(j * L, L)] = acc[j]
    pltpu.sync_copy(acc_v, out_h.at[pl.ds(wid, 1)])


_sc_partials = functools.partial(
    pl.kernel,
    mesh=plsc.VectorSubcoreMesh(core_axis_name="c", subcore_axis_name="s"),
    out_type=jax.ShapeDtypeStruct((NW, HID), jnp.float32),
    scratch_types=[
        pltpu.VMEM((PAD,), jnp.int32),
        pltpu.VMEM((L,), jnp.int32),
        pltpu.VMEM((L, HID), jnp.float32),
        pltpu.VMEM((ROWS, HID), jnp.float32),
        pltpu.VMEM((1, HID), jnp.float32),
        pltpu.SemaphoreType.DMA,
        pltpu.SemaphoreType.DMA,
    ],
)(_sc_body)


def _epilogue_body(p_ref, w1l_ref, b1_ref, q_ref, w2t_ref, b2_ref, o_ref):
    h = (jnp.sum(p_ref[...], axis=0, keepdims=True) + b1_ref[...]
         + q_ref[0, 0] * w1l_ref[...])
    h = jnp.maximum(h, 0.0)
    o = jnp.sum(h * w2t_ref[...], axis=1, keepdims=True) + b2_ref[...]
    o_ref[...] = 1.0 / (1.0 + jnp.exp(-o))


def kernel(category, title, quantity, emb, emb2, W1, b1, W2, b2):
    partials = _sc_partials(
        category.astype(jnp.int32), title.astype(jnp.int32),
        emb.reshape(-1, HID), emb2.reshape(-1, HID), W1)
    w1_last = lax.slice(W1, (IN_DIM - 1, 0), (IN_DIM, HID))
    return pl.pallas_call(
        _epilogue_body,
        out_shape=jax.ShapeDtypeStruct((1, 1), jnp.float32),
    )(partials, w1_last, b1.reshape(1, HID), quantity.reshape(1, 1),
      W2.reshape(1, HID), b2.reshape(1, 1))


# speedup vs baseline: 1.6507x; 1.6507x over previous
import functools

import jax
import jax.numpy as jnp
from jax import lax
from jax.experimental import pallas as pl
from jax.experimental.pallas import tpu as pltpu
from jax.experimental.pallas import tpu_sc as plsc

NC = 2
NS = 16
L = 16
NW = NC * NS
DIM = 64
CTX = 200
TOK = CTX + 1
HID = 128
IN_DIM = DIM * TOK + 1
TPW = 7
ROWS = TPW * DIM
YROWS = 208


NBUF = 16


def _gather_body(cat_ref, title_ref, emb_ref, emb2_ref, y_ref, blk_ref, sems):
    y_ref[pl.ds(CTX, YROWS - CTX), :] = jnp.zeros((YROWS - CTX, DIM),
                                                  jnp.float32)

    def row_idx(i):
        return cat_ref[0] if i == 0 else title_ref[i - 1]

    def start(i):
        r = row_idx(i)
        src = emb2_ref if i == 0 else emb_ref
        slot = i % NBUF
        base = pl.multiple_of((r // 8) * 8, 8)
        return pltpu.make_async_copy(
            src.at[pl.ds(base, 8)],
            blk_ref.at[pl.ds(slot * 8, 8)],
            sems.at[slot])

    def drain(i, cp):
        cp.wait()
        sub = row_idx(i) % 8
        blk = blk_ref[pl.ds((i % NBUF) * 8, 8), :]
        oh = lax.broadcasted_iota(jnp.int32, (8, DIM), 0) == sub
        row = jnp.sum(jnp.where(oh, blk, 0.0), axis=0, keepdims=True)
        y_ref[pl.ds(i, 1), :] = row

    cps = {}
    for i in range(TOK):
        if i >= NBUF:
            drain(i - NBUF, cps.pop(i - NBUF))
        cps[i] = start(i)
        cps[i].start()
    for i in range(TOK - NBUF, TOK):
        drain(i, cps.pop(i))


def _gather_rows(category, title, emb, emb2):
    return pl.pallas_call(
        _gather_body,
        out_shape=jax.ShapeDtypeStruct((YROWS, DIM), jnp.float32),
        in_specs=[
            pl.BlockSpec(memory_space=pltpu.SMEM),
            pl.BlockSpec(memory_space=pltpu.SMEM),
            pl.BlockSpec(memory_space=pl.ANY),
            pl.BlockSpec(memory_space=pl.ANY),
        ],
        scratch_shapes=[
            pltpu.VMEM((NBUF * 8, DIM), jnp.float32),
            pltpu.SemaphoreType.DMA((NBUF,)),
        ],
    )(category, title, emb, emb2)


def _sc_body(y_h, w1_h, out_h, rows_v, w1_v, acc_v):
    c = lax.axis_index("c")
    s = lax.axis_index("s")
    wid = s * NC + c
    tb = jnp.minimum(TPW * wid, TOK - TPW)

    tb8 = pl.multiple_of((tb // 8) * 8, 8)
    d0 = tb - tb8
    pltpu.sync_copy(y_h.at[pl.ds(tb8, 2 * 8)], rows_v)
    pltpu.sync_copy(w1_h.at[pl.ds(DIM * tb, ROWS)], w1_v)

    hi = jnp.minimum(TPW * wid + TPW, TOK)
    acc = tuple(jnp.zeros((L,), jnp.float32) for _ in range(HID // L))
    for k in range(TPW):
        t = tb + k
        scale = jnp.where((t >= TPW * wid) & (t < hi), 1.0, 0.0)
        scale = scale.astype(jnp.float32)

        def body(db, a, k=k, scale=scale):
            xv = rows_v[d0 + k, pl.ds(db * L, L)] * scale
            for e in range(L):
                xb = jnp.broadcast_to(xv[e], (L,))
                i = k * DIM + db * L + e
                a = tuple(a[j] + xb * w1_v[i, pl.ds(j * L, L)]
                          for j in range(HID // L))
            return a

        acc = lax.fori_loop(0, DIM // L, body, acc)
    for j in range(HID // L):
        acc_v[0, pl.ds(j * L, L)] = acc[j]
    pltpu.sync_copy(acc_v, out_h.at[pl.ds(wid, 1)])


_sc_partials = functools.partial(
    pl.kernel,
    mesh=plsc.VectorSubcoreMesh(core_axis_name="c", subcore_axis_name="s"),
    out_type=jax.ShapeDtypeStruct((NW, HID), jnp.float32),
    scratch_types=[
        pltpu.VMEM((2 * 8, DIM), jnp.float32),
        pltpu.VMEM((ROWS, HID), jnp.float32),
        pltpu.VMEM((1, HID), jnp.float32),
    ],
)(_sc_body)


def _epilogue_body(p_ref, w1l_ref, b1_ref, q_ref, w2t_ref, b2_ref, o_ref):
    h = (jnp.sum(p_ref[...], axis=0, keepdims=True) + b1_ref[...]
         + q_ref[0, 0] * w1l_ref[...])
    h = jnp.maximum(h, 0.0)
    o = jnp.sum(h * w2t_ref[...], axis=1, keepdims=True) + b2_ref[...]
    o_ref[...] = 1.0 / (1.0 + jnp.exp(-o))


def kernel(category, title, quantity, emb, emb2, W1, b1, W2, b2):
    y = _gather_rows(category.astype(jnp.int32), title.astype(jnp.int32),
                     emb, emb2)
    partials = _sc_partials(y, W1)
    w1_last = lax.slice(W1, (IN_DIM - 1, 0), (IN_DIM, HID))
    return pl.pallas_call(
        _epilogue_body,
        out_shape=jax.ShapeDtypeStruct((1, 1), jnp.float32),
    )(partials, w1_last, b1.reshape(1, HID), quantity.reshape(1, 1),
      W2.reshape(1, HID), b2.reshape(1, 1))


# speedup vs baseline: 12.6503x; 7.6634x over previous
import functools

import jax
import jax.numpy as jnp
from jax import lax
from jax.experimental import pallas as pl
from jax.experimental.pallas import tpu as pltpu
from jax.experimental.pallas import tpu_sc as plsc

NC = 2
NS = 16
L = 16
NW = NC * NS
DIM = 64
CTX = 200
CAT = 1000
TOK = CTX + 1
HID = 128
IN_DIM = DIM * TOK + 1
TPW = 7
ROWS = TPW * DIM
YTOK = 512
NBUF = 8


def _gather_body(cat_ref, title_ref, embt_ref, emb2t_ref, yt_ref,
                 blk_ref, e2_ref, sems, sem2):
    yt_ref[...] = jnp.zeros((DIM, YTOK), jnp.float32)
    cp2 = pltpu.make_async_copy(emb2t_ref, e2_ref, sem2)
    cp2.start()

    def start(i):
        r = title_ref[i - 1]
        base = pl.multiple_of((r // 128) * 128, 128)
        slot = (i - 1) % NBUF
        cp = pltpu.make_async_copy(
            embt_ref.at[:, pl.ds(base, 128)],
            blk_ref.at[:, pl.ds(slot * 128, 128)],
            sems.at[slot])
        cp.start()
        return cp

    def drain(i, cp):
        cp.wait()
        r = title_ref[i - 1]
        slot = (i - 1) % NBUF
        blk = blk_ref[:, pl.ds(slot * 128, 128)]
        oh = lax.broadcasted_iota(jnp.int32, (DIM, 128), 1) == (r % 128)
        col = jnp.sum(jnp.where(oh, blk, 0.0), axis=1, keepdims=True)
        yt_ref[:, pl.ds(16 * (i // TPW) + i % TPW, 1)] = col

    cps = {}
    for i in range(1, TOK):
        if i > NBUF:
            drain(i - NBUF, cps.pop(i - NBUF))
        cps[i] = start(i)
    cp2.wait()
    oh2 = lax.broadcasted_iota(jnp.int32, (DIM, CAT), 1) == cat_ref[0]
    col2 = jnp.sum(jnp.where(oh2, e2_ref[...], 0.0), axis=1, keepdims=True)
    yt_ref[:, pl.ds(0, 1)] = col2
    for i in range(TOK - NBUF, TOK):
        drain(i, cps.pop(i))


def _gather_rows(category, title, embt, emb2t):
    return pl.pallas_call(
        _gather_body,
        out_shape=jax.ShapeDtypeStruct((DIM, YTOK), jnp.float32),
        in_specs=[
            pl.BlockSpec(memory_space=pltpu.SMEM),
            pl.BlockSpec(memory_space=pltpu.SMEM),
            pl.BlockSpec(memory_space=pl.ANY),
            pl.BlockSpec(memory_space=pl.ANY),
        ],
        scratch_shapes=[
            pltpu.VMEM((DIM, NBUF * 128), jnp.float32),
            pltpu.VMEM((DIM, CAT), jnp.float32),
            pltpu.SemaphoreType.DMA((NBUF,)),
            pltpu.SemaphoreType.DMA,
        ],
    )(category, title, embt, emb2t)


def _sc_body(yt_h, w1_h, out_h, yt_v, w1_v, acc_v):
    c = lax.axis_index("c")
    s = lax.axis_index("s")
    wid = s * NC + c
    cb = pl.multiple_of((wid // 8) * 128, 128)
    lb = pl.multiple_of((wid % 8) * L, L)
    pltpu.sync_copy(yt_h.at[:, pl.ds(cb, 128)], yt_v)

    rb = pl.multiple_of(jnp.minimum(ROWS * wid, IN_DIM - 1 - ROWS), 8)
    delta = ROWS * wid - rb
    pltpu.sync_copy(w1_h.at[pl.ds(rb, ROWS)], w1_v)

    scales = []
    for k in range(TPW):
        sc = jnp.where(TPW * wid + k < TOK, 1.0, 0.0)
        scales.append(sc.astype(jnp.float32))

    def body(d, a):
        xrow = yt_v[d, pl.ds(lb, L)]
        for k in range(TPW):
            xb = jnp.broadcast_to(xrow[k], (L,)) * scales[k]
            i = jnp.minimum(k * DIM + delta + d, ROWS - 1)
            a = tuple(a[j] + xb * w1_v[i, pl.ds(j * L, L)]
                      for j in range(HID // L))
        return a

    acc = tuple(jnp.zeros((L,), jnp.float32) for _ in range(HID // L))
    acc = lax.fori_loop(0, DIM, body, acc)
    for j in range(HID // L):
        acc_v[0, pl.ds(j * L, L)] = acc[j]
    pltpu.sync_copy(acc_v, out_h.at[pl.ds(wid, 1)])


_sc_partials = functools.partial(
    pl.kernel,
    mesh=plsc.VectorSubcoreMesh(core_axis_name="c", subcore_axis_name="s"),
    out_type=jax.ShapeDtypeStruct((NW, HID), jnp.float32),
    scratch_types=[
        pltpu.VMEM((DIM, 128), jnp.float32),
        pltpu.VMEM((ROWS, HID), jnp.float32),
        pltpu.VMEM((1, HID), jnp.float32),
    ],
)(_sc_body)


def _epilogue_body(p_ref, w1l_ref, b1_ref, q_ref, w2t_ref, b2_ref, o_ref):
    h = (jnp.sum(p_ref[...], axis=0, keepdims=True) + b1_ref[...]
         + q_ref[0, 0] * w1l_ref[...])
    h = jnp.maximum(h, 0.0)
    o = jnp.sum(h * w2t_ref[...], axis=1, keepdims=True) + b2_ref[...]
    o_ref[...] = 1.0 / (1.0 + jnp.exp(-o))


def kernel(category, title, quantity, emb, emb2, W1, b1, W2, b2):
    yt = _gather_rows(category.astype(jnp.int32), title.astype(jnp.int32),
                      emb.T, emb2.T)
    partials = _sc_partials(yt, W1)
    w1_last = lax.slice(W1, (IN_DIM - 1, 0), (IN_DIM, HID))
    return pl.pallas_call(
        _epilogue_body,
        out_shape=jax.ShapeDtypeStruct((1, 1), jnp.float32),
    )(partials, w1_last, b1.reshape(1, HID), quantity.reshape(1, 1),
      W2.reshape(1, HID), b2.reshape(1, 1))
